# baseline (speedup 1.0000x reference)
import functools

import jax
import jax.numpy as jnp
from jax import lax
from jax.experimental import pallas as pl
from jax.experimental.pallas import tpu as pltpu
from jax.experimental.pallas import tpu_sc as plsc

NC = 2
NS = 16
NW = NC * NS
B = 16384
D = 32
V = 100000
BPW = B // NW
IC = 4096
NIC = B // IC
BCH = 128


UNR = 16
STEP = 16 * UNR


def _sc_body(uembT, pembT, ubias, pbias, idx_h,
             part_out, bsum_out,
             row_v, uvals_v, idxc_v, bidxu_v, bidxp_v,
             ub_v, pb_v, bsum_v, acc_v, sem, rsem, rsem2, isem0, isem1):
    cid = lax.axis_index("c")
    sid = lax.axis_index("s")
    wid = sid * NC + cid

    urow_cps = [pltpu.async_copy(uembT.at[wid], row_v, rsem)]

    uidx_h = idx_h.at[0]
    pidx_h = idx_h.at[1]
    pltpu.sync_copy(uidx_h.at[pl.ds(wid * BPW, BPW)], bidxu_v)
    pltpu.sync_copy(pidx_h.at[pl.ds(wid * BPW, BPW)], bidxp_v)
    bias_copies = []
    for j in range(BPW // BCH):
        bias_copies.append(pltpu.async_copy(
            ubias.at[0].at[bidxu_v.at[pl.ds(j * BCH, BCH)]],
            ub_v.at[pl.ds(j * BCH, BCH)], sem))
        bias_copies.append(pltpu.async_copy(
            pbias.at[0].at[bidxp_v.at[pl.ds(j * BCH, BCH)]],
            pb_v.at[pl.ds(j * BCH, BCH)], sem))

    isems = (isem0, isem1)
    cps = [pltpu.async_copy(uidx_h.at[pl.ds(0, IC)], idxc_v.at[0], isem0),
           pltpu.async_copy(uidx_h.at[pl.ds(IC, IC)], idxc_v.at[1], isem1)]
    for cp in urow_cps:
        cp.wait()

    def u_chunk(c_base, buf):
        @plsc.parallel_loop(0, IC, step=STEP)
        def body(o):
            for t in range(UNR):
                vec_idx = idxc_v[buf, pl.ds(o + t * 16, 16)]
                uvals_v[pl.ds(c_base + o + t * 16, 16)] = plsc.load_gather(
                    row_v, [vec_idx])

    for c in range(NIC):
        buf = c % 2
        cps[buf].wait()
        u_chunk(c * IC, buf)
        nxt = c + 2
        if nxt < NIC:
            cps[buf] = pltpu.async_copy(
                uidx_h.at[pl.ds(nxt * IC, IC)], idxc_v.at[buf], isems[buf])
        elif nxt < 2 * NIC:
            cps[buf] = pltpu.async_copy(
                pidx_h.at[pl.ds((nxt - NIC) * IC, IC)], idxc_v.at[buf],
                isems[buf])

    prow_cps = [pltpu.async_copy(pembT.at[wid], row_v, rsem)]

    for c in bias_copies:
        c.wait()
    for k in range(BPW // 16):
        bsum_v[pl.ds(k * 16, 16)] = (ub_v[pl.ds(k * 16, 16)]
                                     + pb_v[pl.ds(k * 16, 16)])
    pltpu.sync_copy(bsum_v, bsum_out.at[pl.ds(wid * BPW, BPW)])

    for cp in prow_cps:
        cp.wait()

    def p_chunk(c_base, buf, accs):
        @plsc.parallel_loop(0, IC, step=STEP, carry=accs)
        def body(o, a):
            a = list(a)
            for t in range(UNR):
                vec_idx = idxc_v[buf, pl.ds(o + t * 16, 16)]
                pv = plsc.load_gather(row_v, [vec_idx])
                uv = uvals_v[pl.ds(c_base + o + t * 16, 16)]
                a[t] = a[t] + pv * uv
            return tuple(a)
        return body

    accs = tuple(jnp.zeros((16,), jnp.float32) for _ in range(UNR))
    for c in range(NIC):
        buf = c % 2
        cps[buf].wait()
        accs = p_chunk(c * IC, buf, accs)
        nxt = c + 2
        if nxt < NIC:
            cps[buf] = pltpu.async_copy(
                pidx_h.at[pl.ds(nxt * IC, IC)], idxc_v.at[buf], isems[buf])

    acc = accs[0]
    for t in range(1, UNR):
        acc = acc + accs[t]
    acc_v[...] = acc
    pltpu.sync_copy(acc_v, part_out.at[wid])


_sc_gather_reduce = functools.partial(
    pl.kernel,
    mesh=plsc.VectorSubcoreMesh(core_axis_name="c", subcore_axis_name="s"),
    out_type=[
        jax.ShapeDtypeStruct((NW, 16), jnp.float32),
        jax.ShapeDtypeStruct((B,), jnp.float32),
    ],
    scratch_types=[
        pltpu.VMEM((V,), jnp.float32),
        pltpu.VMEM((B,), jnp.float32),
        pltpu.VMEM((2, IC), jnp.int32),
        pltpu.VMEM((BPW,), jnp.int32),
        pltpu.VMEM((BPW,), jnp.int32),
        pltpu.VMEM((BPW,), jnp.float32),
        pltpu.VMEM((BPW,), jnp.float32),
        pltpu.VMEM((BPW,), jnp.float32),
        pltpu.VMEM((16,), jnp.float32),
        pltpu.SemaphoreType.DMA,
        pltpu.SemaphoreType.DMA,
        pltpu.SemaphoreType.DMA,
        pltpu.SemaphoreType.DMA,
        pltpu.SemaphoreType.DMA,
    ],
    compiler_params=pltpu.CompilerParams(use_tc_tiling_on_sc=True,
                                         needs_layout_passes=False),
)(_sc_body)


def _tc_epilogue(part_ref, bsum_ref, out_ref):
    s = jnp.sum(part_ref[...])
    out_ref[...] = jax.nn.sigmoid(bsum_ref[...] + s)


def kernel(inputs, user_emb, product_emb, user_bias, product_bias):
    partials, bsum = _sc_gather_reduce(
        user_emb.T, product_emb.T, user_bias.T, product_bias.T, inputs.T)

    out = pl.pallas_call(
        _tc_epilogue,
        out_shape=jax.ShapeDtypeStruct((128, 128), jnp.float32),
    )(partials, bsum.reshape(128, 128))
    return out.reshape(B, 1)


# speedup vs baseline: 1.0575x; 1.0575x over previous
import functools

import jax
import jax.numpy as jnp
from jax import lax
from jax.experimental import pallas as pl
from jax.experimental.pallas import tpu as pltpu
from jax.experimental.pallas import tpu_sc as plsc

NC = 2
NS = 16
NW = NC * NS
B = 16384
D = 32
V = 100000
BPW = B // NW
IC = 4096
NIC = B // IC
BCH = 128


UNR = 4
STEP = 16 * UNR


def _sc_body(uembT, pembT, ubias, pbias, idx_h,
             part_out, bsum_out,
             row_v, uvals_v, idxc_v, bidxu_v, bidxp_v,
             ub_v, pb_v, bsum_v, acc_v, sem, rsem, rsem2, isem0, isem1):
    cid = lax.axis_index("c")
    sid = lax.axis_index("s")
    wid = sid * NC + cid

    urow_cps = [pltpu.async_copy(uembT.at[wid], row_v, rsem)]

    uidx_h = idx_h.at[0]
    pidx_h = idx_h.at[1]
    pltpu.sync_copy(uidx_h.at[pl.ds(wid * BPW, BPW)], bidxu_v)
    pltpu.sync_copy(pidx_h.at[pl.ds(wid * BPW, BPW)], bidxp_v)
    bias_copies = []
    for j in range(BPW // BCH):
        bias_copies.append(pltpu.async_copy(
            ubias.at[0].at[bidxu_v.at[pl.ds(j * BCH, BCH)]],
            ub_v.at[pl.ds(j * BCH, BCH)], sem))
        bias_copies.append(pltpu.async_copy(
            pbias.at[0].at[bidxp_v.at[pl.ds(j * BCH, BCH)]],
            pb_v.at[pl.ds(j * BCH, BCH)], sem))

    isems = (isem0, isem1)
    cps = [pltpu.async_copy(uidx_h.at[pl.ds(0, IC)], idxc_v.at[0], isem0),
           pltpu.async_copy(uidx_h.at[pl.ds(IC, IC)], idxc_v.at[1], isem1)]
    for cp in urow_cps:
        cp.wait()

    def u_chunk(c_base, buf):
        @plsc.parallel_loop(0, IC, step=STEP)
        def body(o):
            for t in range(UNR):
                vec_idx = idxc_v[buf, pl.ds(o + t * 16, 16)]
                uvals_v[pl.ds(c_base + o + t * 16, 16)] = plsc.load_gather(
                    row_v, [vec_idx])

    for c in range(NIC):
        buf = c % 2
        cps[buf].wait()
        u_chunk(c * IC, buf)
        nxt = c + 2
        if nxt < NIC:
            cps[buf] = pltpu.async_copy(
                uidx_h.at[pl.ds(nxt * IC, IC)], idxc_v.at[buf], isems[buf])
        elif nxt < 2 * NIC:
            cps[buf] = pltpu.async_copy(
                pidx_h.at[pl.ds((nxt - NIC) * IC, IC)], idxc_v.at[buf],
                isems[buf])

    prow_cps = [pltpu.async_copy(pembT.at[wid], row_v, rsem)]

    for c in bias_copies:
        c.wait()
    for k in range(BPW // 16):
        bsum_v[pl.ds(k * 16, 16)] = (ub_v[pl.ds(k * 16, 16)]
                                     + pb_v[pl.ds(k * 16, 16)])
    pltpu.sync_copy(bsum_v, bsum_out.at[pl.ds(wid * BPW, BPW)])

    for cp in prow_cps:
        cp.wait()

    def p_chunk(c_base, buf, accs):
        @plsc.parallel_loop(0, IC, step=STEP, carry=accs)
        def body(o, a):
            a = list(a)
            for t in range(UNR):
                vec_idx = idxc_v[buf, pl.ds(o + t * 16, 16)]
                pv = plsc.load_gather(row_v, [vec_idx])
                uv = uvals_v[pl.ds(c_base + o + t * 16, 16)]
                a[t] = a[t] + pv * uv
            return tuple(a)
        return body

    accs = tuple(jnp.zeros((16,), jnp.float32) for _ in range(UNR))
    for c in range(NIC):
        buf = c % 2
        cps[buf].wait()
        accs = p_chunk(c * IC, buf, accs)
        nxt = c + 2
        if nxt < NIC:
            cps[buf] = pltpu.async_copy(
                pidx_h.at[pl.ds(nxt * IC, IC)], idxc_v.at[buf], isems[buf])

    acc = accs[0]
    for t in range(1, UNR):
        acc = acc + accs[t]
    acc_v[...] = acc
    pltpu.sync_copy(acc_v, part_out.at[wid])


_sc_gather_reduce = functools.partial(
    pl.kernel,
    mesh=plsc.VectorSubcoreMesh(core_axis_name="c", subcore_axis_name="s"),
    out_type=[
        jax.ShapeDtypeStruct((NW, 16), jnp.float32),
        jax.ShapeDtypeStruct((B,), jnp.float32),
    ],
    scratch_types=[
        pltpu.VMEM((V,), jnp.float32),
        pltpu.VMEM((B,), jnp.float32),
        pltpu.VMEM((2, IC), jnp.int32),
        pltpu.VMEM((BPW,), jnp.int32),
        pltpu.VMEM((BPW,), jnp.int32),
        pltpu.VMEM((BPW,), jnp.float32),
        pltpu.VMEM((BPW,), jnp.float32),
        pltpu.VMEM((BPW,), jnp.float32),
        pltpu.VMEM((16,), jnp.float32),
        pltpu.SemaphoreType.DMA,
        pltpu.SemaphoreType.DMA,
        pltpu.SemaphoreType.DMA,
        pltpu.SemaphoreType.DMA,
        pltpu.SemaphoreType.DMA,
    ],
    compiler_params=pltpu.CompilerParams(use_tc_tiling_on_sc=True,
                                         needs_layout_passes=False),
)(_sc_body)


def _tc_epilogue(part_ref, bsum_ref, out_ref):
    s = jnp.sum(part_ref[...])
    out_ref[...] = jax.nn.sigmoid(bsum_ref[...] + s)


def kernel(inputs, user_emb, product_emb, user_bias, product_bias):
    partials, bsum = _sc_gather_reduce(
        user_emb.T, product_emb.T, user_bias.T, product_bias.T, inputs.T)

    out = pl.pallas_call(
        _tc_epilogue,
        out_shape=jax.ShapeDtypeStruct((128, 128), jnp.float32),
    )(partials, bsum.reshape(128, 128))
    return out.reshape(B, 1)
